# baseline (device time: 255461 ns/iter reference)
import functools

import jax
import jax.numpy as jnp
from jax import lax
from jax.experimental import pallas as pl
from jax.experimental.pallas import tpu as pltpu

N_DEV = 32
SQ = 1024
D_MODEL = 1024
HEADS = 8
DH = 128
SCALE = 0.08838834764831843
CHUNK = SQ // N_DEV


def kernel(x, Wq, K_ext, V_ext, Wo):
    pos = lax.axis_index("i")
    x2 = x[0].astype(jnp.bfloat16)
    wq = lax.dynamic_slice_in_dim(Wq, pos * (HEADS * DH), HEADS * DH, 1)
    wq = wq.astype(jnp.bfloat16)
    k = K_ext[0].astype(jnp.bfloat16).transpose(1, 0, 2)
    v = V_ext[0].astype(jnp.bfloat16).transpose(1, 0, 2)
    wo = lax.dynamic_slice_in_dim(Wo, pos * (HEADS * DH), HEADS * DH, 0)
    wo = wo.astype(jnp.bfloat16)

    def body(x_ref, wq_ref, k_ref, v_ref, wo_ref, out_ref,
             staging, bias_ref, q_ref, ctx_ref,
             rs_send, rs_recv, ag_send, ag_recv):
        my = lax.axis_index("i")
        left = lax.rem(my - 1 + N_DEV, N_DEV)
        right = lax.rem(my + 1, N_DEV)

        barrier_sem = pltpu.get_barrier_semaphore()
        for nbr in (left, right):
            pl.semaphore_signal(
                barrier_sem, inc=1,
                device_id=(nbr,), device_id_type=pl.DeviceIdType.MESH,
            )
        pl.semaphore_wait(barrier_sem, 2)

        q_ref[:, :] = lax.dot_general(
            x_ref[:, :], wq_ref[:, :], (((1,), (0,)), ((), ())),
            preferred_element_type=jnp.float32,
        ).astype(jnp.bfloat16)

        ri = lax.broadcasted_iota(jnp.int32, (SQ, SQ), 0) // 64
        ci = lax.broadcasted_iota(jnp.int32, (SQ, SQ), 1) // 64
        mask = (ri == ci) | (ci == 0) | (lax.rem(ri + ci, 3) == 0)
        bias_ref[:, :] = jnp.where(mask, 0.0, -1e9).astype(jnp.float32)

        for h in range(HEADS):
            q_h = q_ref[:, h * DH:(h + 1) * DH]
            k_h = k_ref[h]
            scores = lax.dot_general(
                q_h, k_h, (((1,), (1,)), ((), ())),
                preferred_element_type=jnp.float32,
            ) * SCALE + bias_ref[:, :]
            m = jnp.max(scores, axis=-1, keepdims=True)
            w = jnp.exp(scores - m)
            s = jnp.sum(w, axis=-1, keepdims=True)
            p = (w / s).astype(jnp.bfloat16)
            ctx_h = lax.dot_general(
                p, v_ref[h], (((1,), (0,)), ((), ())),
                preferred_element_type=jnp.float32,
            )
            ctx_ref[:, h * DH:(h + 1) * DH] = ctx_h.astype(jnp.bfloat16)

        out_ref[:, :] = lax.dot_general(
            ctx_ref[:, :], wo_ref[:, :], (((1,), (0,)), ((), ())),
            preferred_element_type=jnp.float32,
        )

        for s in range(N_DEV - 1):
            send_c = lax.rem(my - s + N_DEV, N_DEV)
            recv_c = lax.rem(my - s - 1 + N_DEV, N_DEV)
            rdma = pltpu.make_async_remote_copy(
                src_ref=out_ref.at[pl.ds(send_c * CHUNK, CHUNK), :],
                dst_ref=staging.at[s],
                send_sem=rs_send.at[s],
                recv_sem=rs_recv.at[s],
                device_id=(right,),
                device_id_type=pl.DeviceIdType.MESH,
            )
            rdma.start()
            rdma.wait()
            out_ref[pl.ds(recv_c * CHUNK, CHUNK), :] = (
                out_ref[pl.ds(recv_c * CHUNK, CHUNK), :] + staging[s]
            )

        for t in range(N_DEV - 1):
            send_c = lax.rem(my + 1 - t + N_DEV, N_DEV)
            rdma = pltpu.make_async_remote_copy(
                src_ref=out_ref.at[pl.ds(send_c * CHUNK, CHUNK), :],
                dst_ref=out_ref.at[pl.ds(send_c * CHUNK, CHUNK), :],
                send_sem=ag_send.at[t],
                recv_sem=ag_recv.at[t],
                device_id=(right,),
                device_id_type=pl.DeviceIdType.MESH,
            )
            rdma.start()
            rdma.wait()

        @functools.partial(
            pl.run_scoped, second_barrier=pltpu.SemaphoreType.REGULAR
        )
        def _(second_barrier):
            for nbr in (left, right):
                pl.semaphore_signal(
                    second_barrier, inc=1,
                    device_id=(nbr,), device_id_type=pl.DeviceIdType.MESH,
                )
            pl.semaphore_wait(second_barrier, 2)

    out = pl.pallas_call(
        body,
        out_shape=jax.ShapeDtypeStruct((SQ, D_MODEL), jnp.float32),
        in_specs=[pl.BlockSpec(memory_space=pltpu.VMEM)] * 5,
        out_specs=pl.BlockSpec(memory_space=pltpu.VMEM),
        scratch_shapes=[
            pltpu.VMEM((N_DEV - 1, CHUNK, D_MODEL), jnp.float32),
            pltpu.VMEM((SQ, SQ), jnp.float32),
            pltpu.VMEM((SQ, HEADS * DH), jnp.bfloat16),
            pltpu.VMEM((SQ, HEADS * DH), jnp.bfloat16),
            pltpu.SemaphoreType.DMA((N_DEV - 1,)),
            pltpu.SemaphoreType.DMA((N_DEV - 1,)),
            pltpu.SemaphoreType.DMA((N_DEV - 1,)),
            pltpu.SemaphoreType.DMA((N_DEV - 1,)),
        ],
        compiler_params=pltpu.CompilerParams(
            collective_id=0,
            vmem_limit_bytes=100 * 1024 * 1024,
        ),
    )(x2, wq, k, v, wo)
    return out[None]
